# Initial kernel scaffold; baseline (speedup 1.0000x reference)
import jax
import jax.numpy as jnp
from jax.experimental import pallas as pl


def kernel(x, edge_index, batch, Wl0, bl0, Wr0, Wl1, bl1, Wr1, Wlin0, blin0, Wout, bout):
    raise NotImplementedError("write your pallas kernel here")


# speedup vs baseline: 2.7674x; 2.7674x over previous
import functools

import jax
import jax.numpy as jnp
from jax import lax
from jax.experimental import pallas as pl
from jax.experimental.pallas import tpu as pltpu
from jax.experimental.pallas import tpu_sc as plsc

_N = 10000
_D = 128
_E = 320000
_LANES = 128
_EPAD = 327680
_CHUNKS = _EPAD // _LANES
_NC = 2
_NS = 16
_ROWS_T = _CHUNKS // (_NC * _NS)
_NPAD = 10240
_RPS = _NPAD // _NS
_CREP = 16
_GRP = 8
_BLK = 256
_GRID = _NPAD // _BLK

_mesh = plsc.VectorSubcoreMesh(core_axis_name="c", subcore_axis_name="s")


def _sc_body(with_cnt, *refs):
    if with_cnt:
        (table, src1d, dst1d, parts_out, cnt_out,
         acc_sh, cnt_sh, srcb, dstb, rows, onesb, cvec) = refs
    else:
        table, src1d, dst1d, parts_out, acc_sh, srcb, dstb, rows = refs

    cid = lax.axis_index("c")
    sid = lax.axis_index("s")
    base = sid * _RPS

    def _zrow(i, c):
        for j in range(_D // 16):
            rows[i, pl.ds(j * 16, 16)] = jnp.zeros((16,), jnp.float32)
        return c
    lax.fori_loop(0, _LANES, _zrow, 0)
    for k in range(_RPS // _LANES):
        pltpu.sync_copy(rows, acc_sh.at[pl.ds(base + k * _LANES, _LANES)])
    if with_cnt:
        def _zc(i, c):
            cvec[pl.ds(i * 16, 16)] = jnp.zeros((16,), jnp.float32)
            return c
        lax.fori_loop(0, _RPS // 16, _zc, 0)
        pltpu.sync_copy(cvec, cnt_sh.at[pl.ds(base, _RPS)])
        def _oc(i, c):
            onesb[pl.ds(i * 16, 16)] = jnp.ones((16,), jnp.float32)
            return c
        lax.fori_loop(0, _LANES // 16, _oc, 0)
    plsc.subcore_barrier()

    ebase = (cid * _NS + sid) * _ROWS_T * _LANES

    def _step(j, c):
        off = ebase + j * _LANES
        pltpu.sync_copy(src1d.at[pl.ds(off, _LANES)], srcb)
        pltpu.sync_copy(dst1d.at[pl.ds(off, _LANES)], dstb)
        pltpu.sync_copy(table.at[srcb], rows)
        pltpu.sync_copy(rows, acc_sh.at[dstb], add=True)
        if with_cnt:
            pltpu.sync_copy(onesb, cnt_sh.at[dstb], add=True)
        return c
    lax.fori_loop(0, _ROWS_T, _step, 0)

    plsc.subcore_barrier()

    obase = cid * _NPAD + base
    for k in range(_RPS // _LANES):
        pltpu.sync_copy(acc_sh.at[pl.ds(base + k * _LANES, _LANES)], rows)
        pltpu.sync_copy(rows, parts_out.at[pl.ds(obase + k * _LANES, _LANES)])
    if with_cnt:
        pltpu.sync_copy(cnt_sh.at[pl.ds(base, _RPS)], cvec)
        pltpu.sync_copy(cvec, cnt_out.at[pl.ds(obase, _RPS)])


def _make_sc(with_cnt):
    out_type = [jax.ShapeDtypeStruct((_NC * _NPAD, _D), jnp.float32)]
    scratch = [
        pltpu.VMEM_SHARED((_NPAD, _D), jnp.float32),
    ]
    if with_cnt:
        out_type.append(jax.ShapeDtypeStruct((_NC * _NPAD,), jnp.float32))
        scratch.append(pltpu.VMEM_SHARED((_NPAD,), jnp.float32))
    scratch += [
        pltpu.VMEM((_LANES,), jnp.int32),
        pltpu.VMEM((_LANES,), jnp.int32),
        pltpu.VMEM((_LANES, _D), jnp.float32),
    ]
    if with_cnt:
        scratch.append(pltpu.VMEM((_LANES,), jnp.float32))
        scratch.append(pltpu.VMEM((_RPS,), jnp.float32))
    return pl.kernel(
        functools.partial(_sc_body, with_cnt),
        out_type=tuple(out_type) if with_cnt else out_type[0],
        mesh=_mesh,
        scratch_types=tuple(scratch),
    )


_sc_agg_cnt = _make_sc(True)
_sc_agg = _make_sc(False)


def _l0_body(parts, cnt, x, wl, wr, b, h_out):
    a = parts[0] + parts[1]
    c = cnt[0] + cnt[1]
    inv = (1.0 / jnp.maximum(c, 1.0))[:, None]
    h = (jnp.dot(a * inv, wl[...], preferred_element_type=jnp.float32) + b[...]
         + jnp.dot(x[...], wr[...], preferred_element_type=jnp.float32))
    h_out[...] = jnp.maximum(h, 0.0)


def _l1_body(parts, cnt, h0, wl, wr, b, wg, bg, wo, bo, out, acc):
    i = pl.program_id(0)
    a = parts[0] + parts[1]
    c = cnt[0] + cnt[1]
    inv = (1.0 / jnp.maximum(c, 1.0))[:, None]
    h = (jnp.dot(a * inv, wl[...], preferred_element_type=jnp.float32) + b[...]
         + jnp.dot(h0[...], wr[...], preferred_element_type=jnp.float32))
    h = jnp.maximum(h, 0.0)
    rid = lax.broadcasted_iota(jnp.int32, (_BLK, 1), 0) + i * _BLK
    h = jnp.where(rid < _N, h, 0.0)
    s = jnp.sum(h, axis=0, keepdims=True)

    @pl.when(i == 0)
    def _():
        acc[...] = s

    @pl.when(i > 0)
    def _():
        acc[...] = acc[...] + s

    g = acc[...] * (1.0 / _N)
    z = jnp.maximum(
        jnp.dot(g, wg[...], preferred_element_type=jnp.float32) + bg[...], 0.0)
    out[...] = jnp.dot(z, wo[...], preferred_element_type=jnp.float32) + bo[...]


def _l0_call(parts, cnt, x, wlT, wrT, bl):
    return pl.pallas_call(
        _l0_body,
        grid=(_GRID,),
        in_specs=[
            pl.BlockSpec((_NC, _BLK, _D), lambda i: (0, i, 0)),
            pl.BlockSpec((_NC, _BLK), lambda i: (0, i)),
            pl.BlockSpec((_BLK, _D), lambda i: (i, 0)),
            pl.BlockSpec((_D, _D), lambda i: (0, 0)),
            pl.BlockSpec((_D, _D), lambda i: (0, 0)),
            pl.BlockSpec((1, _D), lambda i: (0, 0)),
        ],
        out_specs=pl.BlockSpec((_BLK, _D), lambda i: (i, 0)),
        out_shape=jax.ShapeDtypeStruct((_NPAD, _D), jnp.float32),
    )(parts, cnt, x, wlT, wrT, bl)


def _l1_call(parts, cnt, h0, wlT, wrT, bl, wgT, bg, woT, bo):
    return pl.pallas_call(
        _l1_body,
        grid=(_GRID,),
        in_specs=[
            pl.BlockSpec((_NC, _BLK, _D), lambda i: (0, i, 0)),
            pl.BlockSpec((_NC, _BLK), lambda i: (0, i)),
            pl.BlockSpec((_BLK, _D), lambda i: (i, 0)),
            pl.BlockSpec((_D, _D), lambda i: (0, 0)),
            pl.BlockSpec((_D, _D), lambda i: (0, 0)),
            pl.BlockSpec((1, _D), lambda i: (0, 0)),
            pl.BlockSpec((_D, _D), lambda i: (0, 0)),
            pl.BlockSpec((1, _D), lambda i: (0, 0)),
            pl.BlockSpec((_D, 16), lambda i: (0, 0)),
            pl.BlockSpec((1, 16), lambda i: (0, 0)),
        ],
        out_specs=pl.BlockSpec((1, 16), lambda i: (0, 0)),
        out_shape=jax.ShapeDtypeStruct((1, 16), jnp.float32),
        scratch_shapes=[pltpu.VMEM((1, _D), jnp.float32)],
    )(parts, cnt, h0, wlT, wrT, bl, wgT, bg, woT, bo)


def kernel(x, edge_index, batch, Wl0, bl0, Wr0, Wl1, bl1, Wr1, Wlin0, blin0, Wout, bout):
    src = edge_index[0]
    dst = edge_index[1]
    pad_e = _EPAD - _E
    srcp = jnp.concatenate([src, jnp.zeros((pad_e,), jnp.int32)])
    dstp = jnp.concatenate([dst, jnp.full((pad_e,), _N, jnp.int32)])
    xp = jnp.pad(x, ((0, _NPAD - _N), (0, 0)))

    parts0, cnt = _sc_agg_cnt(xp, srcp, dstp)
    parts0 = parts0.reshape(_NC, _NPAD, _D)
    cnt = cnt.reshape(_NC, _NPAD)
    h0 = _l0_call(parts0, cnt, xp, Wl0.T, Wr0.T, bl0[None, :])
    parts1 = _sc_agg(h0, srcp, dstp).reshape(_NC, _NPAD, _D)
    out = _l1_call(parts1, cnt, h0, Wl1.T, Wr1.T, bl1[None, :],
                   Wlin0.T, blin0[None, :], Wout.T, bout[None, :])
    return out


# speedup vs baseline: 5.9905x; 2.1647x over previous
import functools

import jax
import jax.numpy as jnp
from jax import lax
from jax.experimental import pallas as pl
from jax.experimental.pallas import tpu as pltpu
from jax.experimental.pallas import tpu_sc as plsc

_N = 10000
_D = 128
_E = 320000
_LANES = 128
_EPAD = 327680
_CHUNKS = _EPAD // _LANES
_NC = 2
_NS = 16
_ROWS_T = _CHUNKS // (_NC * _NS)
_NPAD = 10240
_RPS = _NPAD // _NS
_CREP = 16
_GRP = 8
_BLK = 256
_GRID = _NPAD // _BLK

_mesh = plsc.VectorSubcoreMesh(core_axis_name="c", subcore_axis_name="s")


def _sc_body(with_cnt, *refs):
    if with_cnt:
        (table, src1d, dst1d, parts_out, cnt_out,
         acc_sh, cnt_sh, srcb, dstb, rows, onesb, cvec) = refs
    else:
        table, src1d, dst1d, parts_out, acc_sh, srcb, dstb, rows = refs

    cid = lax.axis_index("c")
    sid = lax.axis_index("s")
    base = sid * _RPS

    def _zrow(i, c):
        for j in range(_D // 16):
            rows[i, pl.ds(j * 16, 16)] = jnp.zeros((16,), jnp.float32)
        return c
    lax.fori_loop(0, _LANES, _zrow, 0)
    for k in range(_RPS // _LANES):
        pltpu.sync_copy(rows, acc_sh.at[pl.ds(base + k * _LANES, _LANES)])
    if with_cnt:
        def _zc(i, c):
            cvec[pl.ds(i * 16, 16)] = jnp.zeros((16,), jnp.float32)
            return c
        lax.fori_loop(0, _RPS // 16, _zc, 0)
        pltpu.sync_copy(cvec, cnt_sh.at[pl.ds(base, _RPS)])
        def _oc(i, c):
            onesb[pl.ds(i * 16, 16)] = jnp.ones((16,), jnp.float32)
            return c
        lax.fori_loop(0, _LANES // 16, _oc, 0)
    plsc.subcore_barrier()

    ebase = (cid * _NS + sid) * _ROWS_T * _LANES

    def _step(j, c):
        off = ebase + j * _LANES
        pltpu.sync_copy(src1d.at[pl.ds(off, _LANES)], srcb)
        pltpu.sync_copy(dst1d.at[pl.ds(off, _LANES)], dstb)
        pltpu.sync_copy(table.at[srcb], rows)
        pltpu.sync_copy(rows, acc_sh.at[dstb], add=True)
        if with_cnt:
            pltpu.sync_copy(onesb, cnt_sh.at[dstb], add=True)
        return c
    lax.fori_loop(0, _ROWS_T, _step, 0)

    plsc.subcore_barrier()

    obase = cid * _NPAD + base
    for k in range(_RPS // _LANES):
        pltpu.sync_copy(acc_sh.at[pl.ds(base + k * _LANES, _LANES)], rows)
        pltpu.sync_copy(rows, parts_out.at[pl.ds(obase + k * _LANES, _LANES)])
    if with_cnt:
        pltpu.sync_copy(cnt_sh.at[pl.ds(base, _RPS)], cvec)
        pltpu.sync_copy(cvec, cnt_out.at[pl.ds(obase, _RPS)])


def _make_sc(with_cnt):
    out_type = [jax.ShapeDtypeStruct((_NC * _NPAD, _D), jnp.float32)]
    scratch = [
        pltpu.VMEM_SHARED((_NPAD, _D), jnp.float32),
    ]
    if with_cnt:
        out_type.append(jax.ShapeDtypeStruct((_NC * _NPAD,), jnp.float32))
        scratch.append(pltpu.VMEM_SHARED((_NPAD,), jnp.float32))
    scratch += [
        pltpu.VMEM((_LANES,), jnp.int32),
        pltpu.VMEM((_LANES,), jnp.int32),
        pltpu.VMEM((_LANES, _D), jnp.float32),
    ]
    if with_cnt:
        scratch.append(pltpu.VMEM((_LANES,), jnp.float32))
        scratch.append(pltpu.VMEM((_RPS,), jnp.float32))
    return pl.kernel(
        functools.partial(_sc_body, with_cnt),
        out_type=tuple(out_type) if with_cnt else out_type[0],
        mesh=_mesh,
        scratch_types=tuple(scratch),
    )


_sc_agg_cnt = _make_sc(True)
_sc_agg = _make_sc(False)


def _l0_body(parts, cnt, x, wl, wr, b, h_out):
    a = parts[0] + parts[1]
    c = cnt[0] + cnt[1]
    inv = (1.0 / jnp.maximum(c, 1.0))[:, None]
    h = (jnp.dot(a * inv, wl[...], preferred_element_type=jnp.float32) + b[...]
         + jnp.dot(x[...], wr[...], preferred_element_type=jnp.float32))
    h_out[...] = jnp.maximum(h, 0.0)


def _l1_body(parts, cnt, h0, wl, wr, b, wg, bg, wo, bo, out, acc):
    i = pl.program_id(0)
    a = parts[0] + parts[1]
    c = cnt[0] + cnt[1]
    inv = (1.0 / jnp.maximum(c, 1.0))[:, None]
    h = (jnp.dot(a * inv, wl[...], preferred_element_type=jnp.float32) + b[...]
         + jnp.dot(h0[...], wr[...], preferred_element_type=jnp.float32))
    h = jnp.maximum(h, 0.0)
    rid = lax.broadcasted_iota(jnp.int32, (_BLK, 1), 0) + i * _BLK
    h = jnp.where(rid < _N, h, 0.0)
    s = jnp.sum(h, axis=0, keepdims=True)

    @pl.when(i == 0)
    def _():
        acc[...] = s

    @pl.when(i > 0)
    def _():
        acc[...] = acc[...] + s

    g = acc[...] * (1.0 / _N)
    z = jnp.maximum(
        jnp.dot(g, wg[...], preferred_element_type=jnp.float32) + bg[...], 0.0)
    out[...] = jnp.dot(z, wo[...], preferred_element_type=jnp.float32) + bo[...]


def _l0_call(parts, cnt, x, wlT, wrT, bl):
    return pl.pallas_call(
        _l0_body,
        grid=(_GRID,),
        in_specs=[
            pl.BlockSpec((_NC, _BLK, _D), lambda i: (0, i, 0)),
            pl.BlockSpec((_NC, _BLK), lambda i: (0, i)),
            pl.BlockSpec((_BLK, _D), lambda i: (i, 0)),
            pl.BlockSpec((_D, _D), lambda i: (0, 0)),
            pl.BlockSpec((_D, _D), lambda i: (0, 0)),
            pl.BlockSpec((1, _D), lambda i: (0, 0)),
        ],
        out_specs=pl.BlockSpec((_BLK, _D), lambda i: (i, 0)),
        out_shape=jax.ShapeDtypeStruct((_NPAD, _D), jnp.float32),
    )(parts, cnt, x, wlT, wrT, bl)


def _l1_call(parts, cnt, h0, wlT, wrT, bl, wgT, bg, woT, bo):
    return pl.pallas_call(
        _l1_body,
        grid=(_GRID,),
        in_specs=[
            pl.BlockSpec((_NC, _BLK, _D), lambda i: (0, i, 0)),
            pl.BlockSpec((_NC, _BLK), lambda i: (0, i)),
            pl.BlockSpec((_BLK, _D), lambda i: (i, 0)),
            pl.BlockSpec((_D, _D), lambda i: (0, 0)),
            pl.BlockSpec((_D, _D), lambda i: (0, 0)),
            pl.BlockSpec((1, _D), lambda i: (0, 0)),
            pl.BlockSpec((_D, _D), lambda i: (0, 0)),
            pl.BlockSpec((1, _D), lambda i: (0, 0)),
            pl.BlockSpec((_D, 16), lambda i: (0, 0)),
            pl.BlockSpec((1, 16), lambda i: (0, 0)),
        ],
        out_specs=pl.BlockSpec((1, 16), lambda i: (0, 0)),
        out_shape=jax.ShapeDtypeStruct((1, 16), jnp.float32),
        scratch_shapes=[pltpu.VMEM((1, _D), jnp.float32)],
    )(parts, cnt, h0, wlT, wrT, bl, wgT, bg, woT, bo)


def kernel(x, edge_index, batch, Wl0, bl0, Wr0, Wl1, bl1, Wr1, Wlin0, blin0, Wout, bout):
    src = edge_index[0]
    dst = edge_index[1]
    pad_e = _EPAD - _E
    ar = jnp.arange(pad_e, dtype=jnp.int32)
    srcp = jnp.concatenate([src, ar % _N])
    dstp = jnp.concatenate([dst, _N + ar % (_NPAD - _N)])
    xp = jnp.pad(x, ((0, _NPAD - _N), (0, 0)))

    parts0, cnt = _sc_agg_cnt(xp, srcp, dstp)
    parts0 = parts0.reshape(_NC, _NPAD, _D)
    cnt = cnt.reshape(_NC, _NPAD)
    h0 = _l0_call(parts0, cnt, xp, Wl0.T, Wr0.T, bl0[None, :])
    parts1 = _sc_agg(h0, srcp, dstp).reshape(_NC, _NPAD, _D)
    out = _l1_call(parts1, cnt, h0, Wl1.T, Wr1.T, bl1[None, :],
                   Wlin0.T, blin0[None, :], Wout.T, bout[None, :])
    return out


# speedup vs baseline: 8.7103x; 1.4540x over previous
import functools

import jax
import jax.numpy as jnp
from jax import lax
from jax.experimental import pallas as pl
from jax.experimental.pallas import tpu as pltpu
from jax.experimental.pallas import tpu_sc as plsc

_N = 10000
_D = 128
_E = 320000
_LANES = 112
_ROWS_T = 90
_NC = 2
_NS = 16
_EPAD = _NC * _NS * _ROWS_T * _LANES
_NPAD = 10240
_RPS = _NPAD // _NS
_CP = 80
_BLK = 256
_GRID = _NPAD // _BLK

_mesh = plsc.VectorSubcoreMesh(core_axis_name="c", subcore_axis_name="s")


def _sc_body(with_cnt, *refs):
    if with_cnt:
        (table, src1d, dst1d, parts_out, cnt_out,
         acc_sh, cnt_sh, srcb0, dstb0, srcb1, dstb1, rows0, rows1,
         onesb, cvec, sem0, sem1) = refs
    else:
        (table, src1d, dst1d, parts_out,
         acc_sh, srcb0, dstb0, srcb1, dstb1, rows0, rows1,
         sem0, sem1) = refs

    cid = lax.axis_index("c")
    sid = lax.axis_index("s")
    base = sid * _RPS

    def _zrow(i, c):
        for j in range(_D // 16):
            rows0[i, pl.ds(j * 16, 16)] = jnp.zeros((16,), jnp.float32)
        return c
    lax.fori_loop(0, _CP, _zrow, 0)
    zsrc = rows0.at[pl.ds(0, _CP)]
    for k in range(_RPS // _CP):
        pltpu.sync_copy(zsrc, acc_sh.at[pl.ds(base + k * _CP, _CP)])
    if with_cnt:
        def _zc(i, c):
            cvec[pl.ds(i * 16, 16)] = jnp.zeros((16,), jnp.float32)
            return c
        lax.fori_loop(0, _RPS // 16, _zc, 0)
        pltpu.sync_copy(cvec, cnt_sh.at[pl.ds(base, _RPS)])
        def _oc(i, c):
            onesb[pl.ds(i * 16, 16)] = jnp.ones((16,), jnp.float32)
            return c
        lax.fori_loop(0, _LANES // 16, _oc, 0)
    plsc.subcore_barrier()

    ebase = (cid * _NS + sid) * _ROWS_T * _LANES

    def _load(j, sb, db):
        off = ebase + j * _LANES
        pltpu.sync_copy(src1d.at[pl.ds(off, _LANES)], sb)
        pltpu.sync_copy(dst1d.at[pl.ds(off, _LANES)], db)

    _load(0, srcb0, dstb0)
    pltpu.async_copy(table.at[srcb0], rows0, sem0)

    def _pair(g, c):
        _load(2 * g + 1, srcb1, dstb1)
        pltpu.async_copy(table.at[srcb1], rows1, sem1)
        pltpu.make_async_copy(table.at[srcb0], rows0, sem0).wait()
        pltpu.sync_copy(rows0, acc_sh.at[dstb0], add=True)
        if with_cnt:
            pltpu.sync_copy(onesb, cnt_sh.at[dstb0], add=True)
        _load(2 * g + 2, srcb0, dstb0)
        pltpu.async_copy(table.at[srcb0], rows0, sem0)
        pltpu.make_async_copy(table.at[srcb1], rows1, sem1).wait()
        pltpu.sync_copy(rows1, acc_sh.at[dstb1], add=True)
        if with_cnt:
            pltpu.sync_copy(onesb, cnt_sh.at[dstb1], add=True)
        return c
    lax.fori_loop(0, _ROWS_T // 2, _pair, 0)
    pltpu.make_async_copy(table.at[srcb0], rows0, sem0).wait()

    plsc.subcore_barrier()

    obase = cid * _NPAD + base
    for k in range(_RPS // _CP):
        pltpu.sync_copy(acc_sh.at[pl.ds(base + k * _CP, _CP)], zsrc)
        pltpu.sync_copy(zsrc, parts_out.at[pl.ds(obase + k * _CP, _CP)])
    if with_cnt:
        pltpu.sync_copy(cnt_sh.at[pl.ds(base, _RPS)], cvec)
        pltpu.sync_copy(cvec, cnt_out.at[pl.ds(obase, _RPS)])


def _make_sc(with_cnt):
    out_type = [jax.ShapeDtypeStruct((_NC * _NPAD, _D), jnp.float32)]
    scratch = [
        pltpu.VMEM_SHARED((_NPAD, _D), jnp.float32),
    ]
    if with_cnt:
        out_type.append(jax.ShapeDtypeStruct((_NC * _NPAD,), jnp.float32))
        scratch.append(pltpu.VMEM_SHARED((_NPAD,), jnp.float32))
    scratch += [
        pltpu.VMEM((_LANES,), jnp.int32),
        pltpu.VMEM((_LANES,), jnp.int32),
        pltpu.VMEM((_LANES,), jnp.int32),
        pltpu.VMEM((_LANES,), jnp.int32),
        pltpu.VMEM((_LANES, _D), jnp.float32),
        pltpu.VMEM((_LANES, _D), jnp.float32),
    ]
    if with_cnt:
        scratch.append(pltpu.VMEM((_LANES,), jnp.float32))
        scratch.append(pltpu.VMEM((_RPS,), jnp.float32))
    scratch.append(pltpu.SemaphoreType.DMA)
    scratch.append(pltpu.SemaphoreType.DMA)
    return pl.kernel(
        functools.partial(_sc_body, with_cnt),
        out_type=tuple(out_type) if with_cnt else out_type[0],
        mesh=_mesh,
        scratch_types=tuple(scratch),
    )


_sc_agg_cnt = _make_sc(True)
_sc_agg = _make_sc(False)


def _l0_body(parts, cnt, x, wl, wr, b, h_out):
    a = parts[0] + parts[1]
    c = cnt[0] + cnt[1]
    inv = (1.0 / jnp.maximum(c, 1.0))[:, None]
    h = (jnp.dot(a * inv, wl[...], preferred_element_type=jnp.float32) + b[...]
         + jnp.dot(x[...], wr[...], preferred_element_type=jnp.float32))
    h_out[...] = jnp.maximum(h, 0.0)


def _l1_body(parts, cnt, h0, wl, wr, b, wg, bg, wo, bo, out, acc):
    i = pl.program_id(0)
    a = parts[0] + parts[1]
    c = cnt[0] + cnt[1]
    inv = (1.0 / jnp.maximum(c, 1.0))[:, None]
    h = (jnp.dot(a * inv, wl[...], preferred_element_type=jnp.float32) + b[...]
         + jnp.dot(h0[...], wr[...], preferred_element_type=jnp.float32))
    h = jnp.maximum(h, 0.0)
    rid = lax.broadcasted_iota(jnp.int32, (_BLK, 1), 0) + i * _BLK
    h = jnp.where(rid < _N, h, 0.0)
    s = jnp.sum(h, axis=0, keepdims=True)

    @pl.when(i == 0)
    def _():
        acc[...] = s

    @pl.when(i > 0)
    def _():
        acc[...] = acc[...] + s

    g = acc[...] * (1.0 / _N)
    z = jnp.maximum(
        jnp.dot(g, wg[...], preferred_element_type=jnp.float32) + bg[...], 0.0)
    out[...] = jnp.dot(z, wo[...], preferred_element_type=jnp.float32) + bo[...]


def _l0_call(parts, cnt, x, wlT, wrT, bl):
    return pl.pallas_call(
        _l0_body,
        grid=(_GRID,),
        in_specs=[
            pl.BlockSpec((_NC, _BLK, _D), lambda i: (0, i, 0)),
            pl.BlockSpec((_NC, _BLK), lambda i: (0, i)),
            pl.BlockSpec((_BLK, _D), lambda i: (i, 0)),
            pl.BlockSpec((_D, _D), lambda i: (0, 0)),
            pl.BlockSpec((_D, _D), lambda i: (0, 0)),
            pl.BlockSpec((1, _D), lambda i: (0, 0)),
        ],
        out_specs=pl.BlockSpec((_BLK, _D), lambda i: (i, 0)),
        out_shape=jax.ShapeDtypeStruct((_NPAD, _D), jnp.float32),
    )(parts, cnt, x, wlT, wrT, bl)


def _l1_call(parts, cnt, h0, wlT, wrT, bl, wgT, bg, woT, bo):
    return pl.pallas_call(
        _l1_body,
        grid=(_GRID,),
        in_specs=[
            pl.BlockSpec((_NC, _BLK, _D), lambda i: (0, i, 0)),
            pl.BlockSpec((_NC, _BLK), lambda i: (0, i)),
            pl.BlockSpec((_BLK, _D), lambda i: (i, 0)),
            pl.BlockSpec((_D, _D), lambda i: (0, 0)),
            pl.BlockSpec((_D, _D), lambda i: (0, 0)),
            pl.BlockSpec((1, _D), lambda i: (0, 0)),
            pl.BlockSpec((_D, _D), lambda i: (0, 0)),
            pl.BlockSpec((1, _D), lambda i: (0, 0)),
            pl.BlockSpec((_D, 16), lambda i: (0, 0)),
            pl.BlockSpec((1, 16), lambda i: (0, 0)),
        ],
        out_specs=pl.BlockSpec((1, 16), lambda i: (0, 0)),
        out_shape=jax.ShapeDtypeStruct((1, 16), jnp.float32),
        scratch_shapes=[pltpu.VMEM((1, _D), jnp.float32)],
    )(parts, cnt, h0, wlT, wrT, bl, wgT, bg, woT, bo)


def kernel(x, edge_index, batch, Wl0, bl0, Wr0, Wl1, bl1, Wr1, Wlin0, blin0, Wout, bout):
    src = edge_index[0]
    dst = edge_index[1]
    pad_e = _EPAD + _LANES - _E
    ar = jnp.arange(pad_e, dtype=jnp.int32)
    srcp = jnp.concatenate([src, ar % _N])
    dstp = jnp.concatenate([dst, _N + ar % (_NPAD - _N)])
    xp = jnp.pad(x, ((0, _NPAD - _N), (0, 0)))

    parts0, cnt = _sc_agg_cnt(xp, srcp, dstp)
    parts0 = parts0.reshape(_NC, _NPAD, _D)
    cnt = cnt.reshape(_NC, _NPAD)
    h0 = _l0_call(parts0, cnt, xp, Wl0.T, Wr0.T, bl0[None, :])
    parts1 = _sc_agg(h0, srcp, dstp).reshape(_NC, _NPAD, _D)
    out = _l1_call(parts1, cnt, h0, Wl1.T, Wr1.T, bl1[None, :],
                   Wlin0.T, blin0[None, :], Wout.T, bout[None, :])
    return out


# speedup vs baseline: 9.8081x; 1.1260x over previous
import functools

import jax
import jax.numpy as jnp
from jax import lax
from jax.experimental import pallas as pl
from jax.experimental.pallas import tpu as pltpu
from jax.experimental.pallas import tpu_sc as plsc

_N = 10000
_D = 128
_E = 320000
_LANES = 112
_ROWS_T = 90
_NC = 2
_NS = 16
_EPAD = _NC * _NS * _ROWS_T * _LANES
_NPAD = 10240
_RPS = _NPAD // _NS
_CP = 80
_BLK = 256
_GRID = _NPAD // _BLK

_mesh = plsc.VectorSubcoreMesh(core_axis_name="c", subcore_axis_name="s")


def _sc_body(with_cnt, *refs):
    if with_cnt:
        (table, src1d, dst1d, parts_out, cnt_out,
         acc_sh, cnt_sh, srcb0, dstb0, srcb1, dstb1, rows0, rows1,
         onesb, cvec, sem0, sem1, semi0, semi1) = refs
    else:
        (table, src1d, dst1d, parts_out,
         acc_sh, srcb0, dstb0, srcb1, dstb1, rows0, rows1,
         sem0, sem1, semi0, semi1) = refs

    cid = lax.axis_index("c")
    sid = lax.axis_index("s")
    base = sid * _RPS

    def _zrow(i, c):
        for j in range(_D // 16):
            rows0[i, pl.ds(j * 16, 16)] = jnp.zeros((16,), jnp.float32)
        return c
    lax.fori_loop(0, _CP, _zrow, 0)
    zsrc = rows0.at[pl.ds(0, _CP)]
    for k in range(_RPS // _CP):
        pltpu.sync_copy(zsrc, acc_sh.at[pl.ds(base + k * _CP, _CP)])
    if with_cnt:
        def _zc(i, c):
            cvec[pl.ds(i * 16, 16)] = jnp.zeros((16,), jnp.float32)
            return c
        lax.fori_loop(0, _RPS // 16, _zc, 0)
        pltpu.sync_copy(cvec, cnt_sh.at[pl.ds(base, _RPS)])
        def _oc(i, c):
            onesb[pl.ds(i * 16, 16)] = jnp.ones((16,), jnp.float32)
            return c
        lax.fori_loop(0, _LANES // 16, _oc, 0)
    plsc.subcore_barrier()

    ebase = (cid * _NS + sid) * _ROWS_T * _LANES

    def _load(j, sb, db, sem):
        off = ebase + jnp.minimum(j, _ROWS_T) * _LANES
        pltpu.async_copy(src1d.at[pl.ds(off, _LANES)], sb, sem)
        pltpu.async_copy(dst1d.at[pl.ds(off, _LANES)], db, sem)

    def _wload(sb, db, sem):
        pltpu.make_async_copy(src1d.at[pl.ds(0, _LANES)], sb, sem).wait()
        pltpu.make_async_copy(dst1d.at[pl.ds(0, _LANES)], db, sem).wait()

    _load(0, srcb0, dstb0, semi0)
    _wload(srcb0, dstb0, semi0)
    pltpu.async_copy(table.at[srcb0], rows0, sem0)
    _load(1, srcb1, dstb1, semi1)

    def _half(rows_a, srcb_a, dstb_a, sem_a, semi_a,
              rows_b, srcb_b, dstb_b, sem_b, semi_b, jnext):
        _wload(srcb_b, dstb_b, semi_b)
        pltpu.make_async_copy(table.at[srcb_a], rows_a, sem_a).wait()
        pltpu.async_copy(table.at[srcb_b], rows_b, sem_b)
        if with_cnt:
            pltpu.sync_copy(onesb, cnt_sh.at[dstb_a], add=True)
        pltpu.sync_copy(rows_a, acc_sh.at[dstb_a], add=True)
        _load(jnext, srcb_a, dstb_a, semi_a)

    def _pair(g, c):
        _half(rows0, srcb0, dstb0, sem0, semi0,
              rows1, srcb1, dstb1, sem1, semi1, 2 * g + 2)
        _half(rows1, srcb1, dstb1, sem1, semi1,
              rows0, srcb0, dstb0, sem0, semi0, 2 * g + 3)
        return c
    lax.fori_loop(0, _ROWS_T // 2, _pair, 0)
    pltpu.make_async_copy(table.at[srcb0], rows0, sem0).wait()
    _wload(srcb1, dstb1, semi1)

    plsc.subcore_barrier()

    obase = cid * _NPAD + base
    for k in range(_RPS // _CP):
        pltpu.sync_copy(acc_sh.at[pl.ds(base + k * _CP, _CP)], zsrc)
        pltpu.sync_copy(zsrc, parts_out.at[pl.ds(obase + k * _CP, _CP)])
    if with_cnt:
        pltpu.sync_copy(cnt_sh.at[pl.ds(base, _RPS)], cvec)
        pltpu.sync_copy(cvec, cnt_out.at[pl.ds(obase, _RPS)])


def _make_sc(with_cnt):
    out_type = [jax.ShapeDtypeStruct((_NC * _NPAD, _D), jnp.float32)]
    scratch = [
        pltpu.VMEM_SHARED((_NPAD, _D), jnp.float32),
    ]
    if with_cnt:
        out_type.append(jax.ShapeDtypeStruct((_NC * _NPAD,), jnp.float32))
        scratch.append(pltpu.VMEM_SHARED((_NPAD,), jnp.float32))
    scratch += [
        pltpu.VMEM((_LANES,), jnp.int32),
        pltpu.VMEM((_LANES,), jnp.int32),
        pltpu.VMEM((_LANES,), jnp.int32),
        pltpu.VMEM((_LANES,), jnp.int32),
        pltpu.VMEM((_LANES, _D), jnp.float32),
        pltpu.VMEM((_LANES, _D), jnp.float32),
    ]
    if with_cnt:
        scratch.append(pltpu.VMEM((_LANES,), jnp.float32))
        scratch.append(pltpu.VMEM((_RPS,), jnp.float32))
    scratch.append(pltpu.SemaphoreType.DMA)
    scratch.append(pltpu.SemaphoreType.DMA)
    scratch.append(pltpu.SemaphoreType.DMA)
    scratch.append(pltpu.SemaphoreType.DMA)
    return pl.kernel(
        functools.partial(_sc_body, with_cnt),
        out_type=tuple(out_type) if with_cnt else out_type[0],
        mesh=_mesh,
        scratch_types=tuple(scratch),
    )


_sc_agg_cnt = _make_sc(True)
_sc_agg = _make_sc(False)


def _l0_body(parts, cnt, x, wl, wr, b, h_out):
    a = parts[0] + parts[1]
    c = cnt[0] + cnt[1]
    inv = (1.0 / jnp.maximum(c, 1.0))[:, None]
    h = (jnp.dot(a * inv, wl[...], preferred_element_type=jnp.float32) + b[...]
         + jnp.dot(x[...], wr[...], preferred_element_type=jnp.float32))
    h_out[...] = jnp.maximum(h, 0.0)


def _l1_body(parts, cnt, h0, wl, wr, b, wg, bg, wo, bo, out, acc):
    i = pl.program_id(0)
    a = parts[0] + parts[1]
    c = cnt[0] + cnt[1]
    inv = (1.0 / jnp.maximum(c, 1.0))[:, None]
    h = (jnp.dot(a * inv, wl[...], preferred_element_type=jnp.float32) + b[...]
         + jnp.dot(h0[...], wr[...], preferred_element_type=jnp.float32))
    h = jnp.maximum(h, 0.0)
    rid = lax.broadcasted_iota(jnp.int32, (_BLK, 1), 0) + i * _BLK
    h = jnp.where(rid < _N, h, 0.0)
    s = jnp.sum(h, axis=0, keepdims=True)

    @pl.when(i == 0)
    def _():
        acc[...] = s

    @pl.when(i > 0)
    def _():
        acc[...] = acc[...] + s

    g = acc[...] * (1.0 / _N)
    z = jnp.maximum(
        jnp.dot(g, wg[...], preferred_element_type=jnp.float32) + bg[...], 0.0)
    out[...] = jnp.dot(z, wo[...], preferred_element_type=jnp.float32) + bo[...]


def _l0_call(parts, cnt, x, wlT, wrT, bl):
    return pl.pallas_call(
        _l0_body,
        grid=(_GRID,),
        in_specs=[
            pl.BlockSpec((_NC, _BLK, _D), lambda i: (0, i, 0)),
            pl.BlockSpec((_NC, _BLK), lambda i: (0, i)),
            pl.BlockSpec((_BLK, _D), lambda i: (i, 0)),
            pl.BlockSpec((_D, _D), lambda i: (0, 0)),
            pl.BlockSpec((_D, _D), lambda i: (0, 0)),
            pl.BlockSpec((1, _D), lambda i: (0, 0)),
        ],
        out_specs=pl.BlockSpec((_BLK, _D), lambda i: (i, 0)),
        out_shape=jax.ShapeDtypeStruct((_NPAD, _D), jnp.float32),
    )(parts, cnt, x, wlT, wrT, bl)


def _l1_call(parts, cnt, h0, wlT, wrT, bl, wgT, bg, woT, bo):
    return pl.pallas_call(
        _l1_body,
        grid=(_GRID,),
        in_specs=[
            pl.BlockSpec((_NC, _BLK, _D), lambda i: (0, i, 0)),
            pl.BlockSpec((_NC, _BLK), lambda i: (0, i)),
            pl.BlockSpec((_BLK, _D), lambda i: (i, 0)),
            pl.BlockSpec((_D, _D), lambda i: (0, 0)),
            pl.BlockSpec((_D, _D), lambda i: (0, 0)),
            pl.BlockSpec((1, _D), lambda i: (0, 0)),
            pl.BlockSpec((_D, _D), lambda i: (0, 0)),
            pl.BlockSpec((1, _D), lambda i: (0, 0)),
            pl.BlockSpec((_D, 16), lambda i: (0, 0)),
            pl.BlockSpec((1, 16), lambda i: (0, 0)),
        ],
        out_specs=pl.BlockSpec((1, 16), lambda i: (0, 0)),
        out_shape=jax.ShapeDtypeStruct((1, 16), jnp.float32),
        scratch_shapes=[pltpu.VMEM((1, _D), jnp.float32)],
    )(parts, cnt, h0, wlT, wrT, bl, wgT, bg, woT, bo)


def kernel(x, edge_index, batch, Wl0, bl0, Wr0, Wl1, bl1, Wr1, Wlin0, blin0, Wout, bout):
    src = edge_index[0]
    dst = edge_index[1]
    pad_e = _EPAD + _LANES - _E
    ar = jnp.arange(pad_e, dtype=jnp.int32)
    srcp = jnp.concatenate([src, ar % _N])
    dstp = jnp.concatenate([dst, _N + ar % (_NPAD - _N)])
    xp = jnp.pad(x, ((0, _NPAD - _N), (0, 0)))

    parts0, cnt = _sc_agg_cnt(xp, srcp, dstp)
    parts0 = parts0.reshape(_NC, _NPAD, _D)
    cnt = cnt.reshape(_NC, _NPAD)
    h0 = _l0_call(parts0, cnt, xp, Wl0.T, Wr0.T, bl0[None, :])
    parts1 = _sc_agg(h0, srcp, dstp).reshape(_NC, _NPAD, _D)
    out = _l1_call(parts1, cnt, h0, Wl1.T, Wr1.T, bl1[None, :],
                   Wlin0.T, blin0[None, :], Wout.T, bout[None, :])
    return out
